# baseline (speedup 1.0000x reference)
import functools

import jax
import jax.numpy as jnp
from jax.experimental import pallas as pl
from jax.experimental.pallas import tpu as pltpu

NUM_EXPERTS = 64
SYSTEM2_THRESHOLD = 0.5
BLOCK_T = 1024


def _router_kernel(x_ref, m_ref, wt_ref, o_ref):
    x = x_ref[...]
    logits = jnp.dot(x, wt_ref[...], preferred_element_type=jnp.float32)
    sq = (m_ref[...] > SYSTEM2_THRESHOLD).astype(jnp.float32)
    col = jax.lax.broadcasted_iota(jnp.int32, logits.shape, 1)
    bias = jnp.where(col == 0, (1.0 - sq) * 5.0, 0.0)
    bias = bias + jnp.where(col == NUM_EXPERTS - 1, sq * 5.0, 0.0)
    o_ref[...] = logits + bias


@functools.partial(jax.jit, static_argnames=())
def kernel(hidden_states, mass, W_gate):
    B, T, C = hidden_states.shape
    n_tok = B * T
    flat_hidden = hidden_states.reshape(n_tok, C)
    flat_mass = mass.reshape(n_tok, 1)
    wt = W_gate.T

    grid = (n_tok // BLOCK_T,)
    return pl.pallas_call(
        _router_kernel,
        grid=grid,
        in_specs=[
            pl.BlockSpec((BLOCK_T, C), lambda i: (i, 0)),
            pl.BlockSpec((BLOCK_T, 1), lambda i: (i, 0)),
            pl.BlockSpec((C, NUM_EXPERTS), lambda i: (0, 0)),
        ],
        out_specs=pl.BlockSpec((BLOCK_T, NUM_EXPERTS), lambda i: (i, 0)),
        out_shape=jax.ShapeDtypeStruct((n_tok, NUM_EXPERTS), jnp.float32),
        compiler_params=pltpu.CompilerParams(
            dimension_semantics=("parallel",),
        ),
    )(flat_hidden, flat_mass, wt)
